# baseline (speedup 1.0000x reference)
import jax
import jax.numpy as jnp
from jax.experimental import pallas as pl

GATE_VALUE = 0.5

TOPK = 2
BLOCK_R = 1024


def _fill_kernel(states_ref, g0_ref, g1_ref, g2_ref, g3_ref, idx_ref,
                 scores_ref, mask_ref):
    states_ref[...] = jnp.zeros(states_ref.shape, dtype=states_ref.dtype)

    @pl.when(pl.program_id(0) == 0)
    def _():
        gate = jnp.full(g0_ref.shape, GATE_VALUE, dtype=g0_ref.dtype)
        g0_ref[...] = gate
        g1_ref[...] = gate
        g2_ref[...] = gate
        g3_ref[...] = gate
        lane = jax.lax.broadcasted_iota(jnp.int32, idx_ref.shape, 1)
        idx_ref[...] = lane % TOPK
        scores_ref[...] = jnp.full(scores_ref.shape, GATE_VALUE,
                                   dtype=scores_ref.dtype)
        mlane = jax.lax.broadcasted_iota(jnp.int32, mask_ref.shape, 1)
        mask_ref[...] = (mlane % 16) < TOPK


def kernel(event, slot_states):
    batch_size, num_slots, slot_dim = slot_states.shape
    rows = batch_size * TOPK
    gr = batch_size * num_slots // 128
    ir = batch_size * TOPK // 128
    grid = (rows // BLOCK_R,)
    zero = lambda i: (0, 0)
    outs = pl.pallas_call(
        _fill_kernel,
        grid=grid,
        out_specs=[
            pl.BlockSpec((BLOCK_R, slot_dim), lambda i: (i, 0)),
            pl.BlockSpec((gr, 128), zero),
            pl.BlockSpec((gr, 128), zero),
            pl.BlockSpec((gr, 128), zero),
            pl.BlockSpec((gr, 128), zero),
            pl.BlockSpec((ir, 128), zero),
            pl.BlockSpec((ir, 128), zero),
            pl.BlockSpec((gr, 128), zero),
        ],
        out_shape=[
            jax.ShapeDtypeStruct((rows, slot_dim), jnp.float32),
            jax.ShapeDtypeStruct((gr, 128), jnp.float32),
            jax.ShapeDtypeStruct((gr, 128), jnp.float32),
            jax.ShapeDtypeStruct((gr, 128), jnp.float32),
            jax.ShapeDtypeStruct((gr, 128), jnp.float32),
            jax.ShapeDtypeStruct((ir, 128), jnp.int32),
            jax.ShapeDtypeStruct((ir, 128), jnp.float32),
            jax.ShapeDtypeStruct((gr, 128), jnp.bool_),
        ],
    )()
    states, g0, g1, g2, g3, idx, scores, mask = outs
    gshape = (batch_size, num_slots)
    return (
        g0.reshape(gshape),
        g1.reshape(gshape),
        g2.reshape(gshape),
        g3.reshape(gshape),
        idx.reshape(batch_size, TOPK),
        scores.reshape(batch_size, TOPK),
        mask.reshape(gshape),
        states.reshape(batch_size, TOPK, slot_dim),
    )


# speedup vs baseline: 2.2872x; 2.2872x over previous
import jax
import jax.numpy as jnp
from jax.experimental import pallas as pl

GATE_VALUE = 0.5

TOPK = 2
BLOCK_B = 512


def _fill_kernel(g0_ref, g1_ref, g2_ref, g3_ref, idx_ref, scores_ref,
                 mask_ref, states_ref):
    gate = jnp.full(g0_ref.shape, GATE_VALUE, dtype=g0_ref.dtype)
    g0_ref[...] = gate
    g1_ref[...] = gate
    g2_ref[...] = gate
    g3_ref[...] = gate
    idx_ref[...] = jax.lax.broadcasted_iota(jnp.int32, idx_ref.shape, 1)
    scores_ref[...] = jnp.full(scores_ref.shape, GATE_VALUE,
                               dtype=scores_ref.dtype)
    col = jax.lax.broadcasted_iota(jnp.int32, mask_ref.shape, 1)
    mask_ref[...] = col < TOPK
    states_ref[...] = jnp.zeros(states_ref.shape, dtype=states_ref.dtype)


def kernel(event, slot_states):
    batch_size, num_slots, slot_dim = slot_states.shape
    grid = (batch_size // BLOCK_B,)
    gspec = pl.BlockSpec((BLOCK_B, num_slots), lambda i: (i, 0))
    kspec = pl.BlockSpec((BLOCK_B, TOPK), lambda i: (i, 0))
    outs = pl.pallas_call(
        _fill_kernel,
        grid=grid,
        out_specs=[
            gspec, gspec, gspec, gspec, kspec, kspec, gspec,
            pl.BlockSpec((BLOCK_B, TOPK, slot_dim), lambda i: (i, 0, 0)),
        ],
        out_shape=[
            jax.ShapeDtypeStruct((batch_size, num_slots), jnp.float32),
            jax.ShapeDtypeStruct((batch_size, num_slots), jnp.float32),
            jax.ShapeDtypeStruct((batch_size, num_slots), jnp.float32),
            jax.ShapeDtypeStruct((batch_size, num_slots), jnp.float32),
            jax.ShapeDtypeStruct((batch_size, TOPK), jnp.int32),
            jax.ShapeDtypeStruct((batch_size, TOPK), jnp.float32),
            jax.ShapeDtypeStruct((batch_size, num_slots), jnp.bool_),
            jax.ShapeDtypeStruct((batch_size, TOPK, slot_dim), jnp.float32),
        ],
    )()
    g0, g1, g2, g3, idx, scores, mask, states = outs
    return (g0, g1, g2, g3, idx, scores, mask, states)
